# baseline (speedup 1.0000x reference)
import functools

import jax
import jax.numpy as jnp
from jax import lax
from jax.experimental import pallas as pl
from jax.experimental.pallas import tpu as pltpu
from jax.experimental.pallas import tpu_sc as plsc

N = 10000
E = 320000
F_IN = 128
C = 256
NUM_CLASSES = 16

NC = 2
NS = 16
L = 16
CH = C // NC
Np = 10240
EL = E + N
B = 112
NBATCH = 186
CHUNK = NBATCH * B
Ep = NS * CHUNK
EPAD = Ep - EL
IR = 6
STRIPE = Np // NS

_mesh = plsc.VectorSubcoreMesh(core_axis_name="c", subcore_axis_name="s")


@functools.partial(
    pl.kernel,
    mesh=_mesh,
    compiler_params=pltpu.CompilerParams(needs_layout_passes=False),
    out_type=jax.ShapeDtypeStruct((NC, Np, CH), jnp.float32),
    scratch_types=[
        pltpu.VMEM((Np,), jnp.float32),
        pltpu.VMEM((Np,), jnp.float32),
        pltpu.VMEM((Np,), jnp.float32),
        pltpu.VMEM((IR, B), jnp.int32),
        pltpu.VMEM((IR, B), jnp.int32),
        pltpu.VMEM((B,), jnp.float32),
        pltpu.VMEM((B, CH), jnp.float32),
        pltpu.VMEM_SHARED((Np,), jnp.float32),
        pltpu.VMEM_SHARED((Np, CH), jnp.float32),
    ] + [pltpu.SemaphoreType.DMA] * (IR + 1),
)
def _sc_gat(src_h, dst_h, als_h, ald_h, xl_h, zn_h, zr_h, agg_h,
            als_v, ald_v, den_v, si_v, di_v, val_v, rows_v,
            den_sh, acc_sh, *sems):
    isem = sems[:IR]
    gsem = sems[IR]
    cid = lax.axis_index("c")
    sid = lax.axis_index("s")
    base_e = sid * CHUNK
    r0 = sid * STRIPE

    pltpu.sync_copy(als_h, als_v)
    pltpu.sync_copy(ald_h, ald_v)
    pltpu.sync_copy(zn_h.at[pl.ds(r0, STRIPE)], den_sh.at[pl.ds(r0, STRIPE)])
    pltpu.sync_copy(zr_h, acc_sh.at[pl.ds(r0, STRIPE)])
    plsc.subcore_barrier()

    def istart(tt, i):
        pltpu.async_copy(src_h.at[pl.ds(base_e + tt * B, B)],
                         si_v.at[i], isem[i])
        pltpu.async_copy(dst_h.at[pl.ds(base_e + tt * B, B)],
                         di_v.at[i], isem[i])

    def iwait(i):
        pltpu.make_async_copy(src_h.at[pl.ds(0, B)], si_v.at[i],
                              isem[i]).wait()
        pltpu.make_async_copy(src_h.at[pl.ds(0, B)], di_v.at[i],
                              isem[i]).wait()

    def _ee(i, k):
        s16 = si_v[i, pl.ds(k, L)]
        d16 = di_v[i, pl.ds(k, L)]
        ee = plsc.load_gather(als_v, [s16]) + plsc.load_gather(ald_v, [d16])
        return jnp.where(ee >= 0.0, ee, ee * 0.2), d16

    def p1_slot(tt, i, prefetch):
        iwait(i)

        @pl.loop(0, B, step=L)
        def _(k):
            ee, _ = _ee(i, k)
            val_v[pl.ds(k, L)] = jnp.exp(ee)

        pltpu.sync_copy(val_v, den_sh.at[di_v.at[i]], add=True)
        if prefetch:
            istart(tt + 3, (i + 3) % IR)

    for tt in range(3):
        istart(tt, tt)

    @pl.loop(0, NBATCH - IR, step=IR)
    def _(t):
        for b6 in range(IR):
            p1_slot(t + b6, b6, True)

    for tt in range(NBATCH - IR, NBATCH):
        p1_slot(tt, tt % IR, tt + 3 < NBATCH)

    plsc.subcore_barrier()
    pltpu.sync_copy(den_sh, den_v)

    def p2_slot(tt, i, prefetch):
        iwait(i)
        cp = pltpu.async_copy(xl_h.at[cid].at[si_v.at[i]], rows_v, gsem)

        @pl.loop(0, B, step=L)
        def _(k):
            ee, d16 = _ee(i, k)
            den16 = plsc.load_gather(den_v, [d16])
            val_v[pl.ds(k, L)] = jnp.exp(ee) / (den16 + 1e-16)

        cp.wait()

        @pl.loop(0, B)
        def _(j):
            a16 = plsc.load_gather(val_v, [jnp.zeros((L,), jnp.int32) + j])
            for f in range(CH // L):
                sl = pl.ds(f * L, L)
                rows_v[j, sl] = rows_v[j, sl] * a16

        pltpu.sync_copy(rows_v, acc_sh.at[di_v.at[i]], add=True)
        if prefetch:
            istart(tt + 3, (i + 3) % IR)

    for tt in range(3):
        istart(tt, tt)

    @pl.loop(0, NBATCH - IR, step=IR)
    def _(t):
        for b6 in range(IR):
            p2_slot(t + b6, b6, True)

    for tt in range(NBATCH - IR, NBATCH):
        p2_slot(tt, tt % IR, tt + 3 < NBATCH)

    plsc.subcore_barrier()
    pltpu.sync_copy(acc_sh.at[pl.ds(r0, STRIPE)],
                    agg_h.at[cid].at[pl.ds(r0, STRIPE)])


def _prep1_k(x_ref, w_ref, as_ref, ad_ref, xl_ref, als_ref, ald_ref):
    xl = jnp.dot(x_ref[...], w_ref[...], preferred_element_type=jnp.float32)
    xl_ref[0] = xl[:, :CH]
    xl_ref[1] = xl[:, CH:]
    als_ref[...] = (xl * as_ref[...][None, :]).sum(axis=1)
    ald_ref[...] = (xl * ad_ref[...][None, :]).sum(axis=1)


def _prep2_k(agg_ref, b_ref, w_ref, as_ref, ad_ref, xl_ref, als_ref, ald_ref):
    h = jnp.concatenate([agg_ref[0], agg_ref[1]], axis=1) + b_ref[...][None, :]
    h = jnp.maximum(h, 0.0)
    xl = jnp.dot(h, w_ref[...], preferred_element_type=jnp.float32)
    xl_ref[0] = xl[:, :CH]
    xl_ref[1] = xl[:, CH:]
    als_ref[...] = (xl * as_ref[...][None, :]).sum(axis=1)
    ald_ref[...] = (xl * ad_ref[...][None, :]).sum(axis=1)


def _head_k(agg_ref, b_ref, mw1_ref, mb1_ref, mw2_ref, mb2_ref, o_ref):
    h = jnp.concatenate([agg_ref[0], agg_ref[1]], axis=1) + b_ref[...][None, :]
    h = jnp.maximum(h, 0.0)
    z = jnp.dot(h, mw1_ref[...], preferred_element_type=jnp.float32)
    z = jnp.maximum(z + mb1_ref[...][None, :], 0.0)
    o = jnp.dot(z, mw2_ref[...], preferred_element_type=jnp.float32)
    o_ref[...] = jax.nn.sigmoid(o + mb2_ref[...][None, :])


_node_out = [
    jax.ShapeDtypeStruct((NC, Np, CH), jnp.float32),
    jax.ShapeDtypeStruct((Np,), jnp.float32),
    jax.ShapeDtypeStruct((Np,), jnp.float32),
]


def kernel(x, edge_index, W1, as1, ad1, b1, W2, as2, ad2, b2,
           mw1, mb1, mw2, mb2):
    ei = edge_index.astype(jnp.int32)
    loop = jnp.arange(N, dtype=jnp.int32)
    pad = jnp.full((EPAD,), N, jnp.int32)
    src = jnp.concatenate([ei[0], loop, pad])
    dst = jnp.concatenate([ei[1], loop, pad])
    x_p = jnp.zeros((Np, F_IN), jnp.float32).at[:N].set(x)
    zn = jnp.zeros((Np,), jnp.float32)
    zr = jnp.zeros((STRIPE, CH), jnp.float32)

    xl1, als1, ald1 = pl.pallas_call(
        _prep1_k, out_shape=_node_out)(x_p, W1, as1.reshape(-1), ad1.reshape(-1))
    agg1 = _sc_gat(src, dst, als1, ald1, xl1, zn, zr)

    xl2, als2, ald2 = pl.pallas_call(
        _prep2_k, out_shape=_node_out)(agg1, b1, W2, as2.reshape(-1),
                                       ad2.reshape(-1))
    agg2 = _sc_gat(src, dst, als2, ald2, xl2, zn, zr)

    out = pl.pallas_call(
        _head_k,
        out_shape=jax.ShapeDtypeStruct((Np, NUM_CLASSES), jnp.float32),
    )(agg2, b2, mw1, mb1, mw2, mb2)
    return out[:N]


# speedup vs baseline: 1.0732x; 1.0732x over previous
import functools

import jax
import jax.numpy as jnp
from jax import lax
from jax.experimental import pallas as pl
from jax.experimental.pallas import tpu as pltpu
from jax.experimental.pallas import tpu_sc as plsc

N = 10000
E = 320000
F_IN = 128
C = 256
NUM_CLASSES = 16

NC = 2
NS = 16
L = 16
CH = C // NC
Np = 10240
EL = E + N
B = 96
NBATCH = 216
CHUNK = NBATCH * B
Ep = NS * CHUNK
EPAD = Ep - EL
IR = 6
STRIPE = Np // NS

_mesh = plsc.VectorSubcoreMesh(core_axis_name="c", subcore_axis_name="s")


@functools.partial(
    pl.kernel,
    mesh=_mesh,
    compiler_params=pltpu.CompilerParams(needs_layout_passes=False),
    out_type=jax.ShapeDtypeStruct((NC, Np, CH), jnp.float32),
    scratch_types=[
        pltpu.VMEM((Np,), jnp.float32),
        pltpu.VMEM((Np,), jnp.float32),
        pltpu.VMEM((Np,), jnp.float32),
        pltpu.VMEM((IR, B), jnp.int32),
        pltpu.VMEM((IR, B), jnp.int32),
        pltpu.VMEM((B,), jnp.float32),
        pltpu.VMEM((B, CH), jnp.float32),
        pltpu.VMEM_SHARED((Np,), jnp.float32),
        pltpu.VMEM_SHARED((Np, CH), jnp.float32),
    ] + [pltpu.SemaphoreType.DMA] * (IR + 1),
)
def _sc_gat(src_h, dst_h, als_h, ald_h, xl_h, zn_h, zr_h, agg_h,
            als_v, ald_v, den_v, si_v, di_v, val_v, rows_v,
            den_sh, acc_sh, *sems):
    isem = sems[:IR]
    gsem = sems[IR]
    cid = lax.axis_index("c")
    sid = lax.axis_index("s")
    base_e = sid * CHUNK
    r0 = sid * STRIPE

    pltpu.sync_copy(als_h, als_v)
    pltpu.sync_copy(ald_h, ald_v)
    pltpu.sync_copy(zn_h.at[pl.ds(r0, STRIPE)], den_sh.at[pl.ds(r0, STRIPE)])
    pltpu.sync_copy(zr_h, acc_sh.at[pl.ds(r0, STRIPE)])
    plsc.subcore_barrier()

    def istart(tt, i):
        pltpu.async_copy(src_h.at[pl.ds(base_e + tt * B, B)],
                         si_v.at[i], isem[i])
        pltpu.async_copy(dst_h.at[pl.ds(base_e + tt * B, B)],
                         di_v.at[i], isem[i])

    def iwait(i):
        pltpu.make_async_copy(src_h.at[pl.ds(0, B)], si_v.at[i],
                              isem[i]).wait()
        pltpu.make_async_copy(src_h.at[pl.ds(0, B)], di_v.at[i],
                              isem[i]).wait()

    def _ee(i, k):
        s16 = si_v[i, pl.ds(k, L)]
        d16 = di_v[i, pl.ds(k, L)]
        ee = plsc.load_gather(als_v, [s16]) + plsc.load_gather(ald_v, [d16])
        return jnp.where(ee >= 0.0, ee, ee * 0.2), d16

    def p1_slot(tt, i, prefetch):
        iwait(i)

        @pl.loop(0, B, step=L)
        def _(k):
            ee, _ = _ee(i, k)
            val_v[pl.ds(k, L)] = jnp.exp(ee)

        pltpu.sync_copy(val_v, den_sh.at[di_v.at[i]], add=True)
        if prefetch:
            istart(tt + 3, (i + 3) % IR)

    for tt in range(3):
        istart(tt, tt)

    @pl.loop(0, NBATCH - IR, step=IR)
    def _(t):
        for b6 in range(IR):
            p1_slot(t + b6, b6, True)

    for tt in range(NBATCH - IR, NBATCH):
        p1_slot(tt, tt % IR, tt + 3 < NBATCH)

    plsc.subcore_barrier()
    pltpu.sync_copy(den_sh, den_v)

    def p2_slot(tt, i, prefetch):
        iwait(i)
        cp = pltpu.async_copy(xl_h.at[cid].at[si_v.at[i]], rows_v, gsem)

        @pl.loop(0, B, step=L)
        def _(k):
            ee, d16 = _ee(i, k)
            den16 = plsc.load_gather(den_v, [d16])
            val_v[pl.ds(k, L)] = jnp.exp(ee) / (den16 + 1e-16)

        cp.wait()

        @pl.loop(0, B)
        def _(j):
            a16 = plsc.load_gather(val_v, [jnp.zeros((L,), jnp.int32) + j])
            for f in range(CH // L):
                sl = pl.ds(f * L, L)
                rows_v[j, sl] = rows_v[j, sl] * a16

        pltpu.sync_copy(rows_v, acc_sh.at[di_v.at[i]], add=True)
        if prefetch:
            istart(tt + 3, (i + 3) % IR)

    for tt in range(3):
        istart(tt, tt)

    @pl.loop(0, NBATCH - IR, step=IR)
    def _(t):
        for b6 in range(IR):
            p2_slot(t + b6, b6, True)

    for tt in range(NBATCH - IR, NBATCH):
        p2_slot(tt, tt % IR, tt + 3 < NBATCH)

    plsc.subcore_barrier()
    pltpu.sync_copy(acc_sh.at[pl.ds(r0, STRIPE)],
                    agg_h.at[cid].at[pl.ds(r0, STRIPE)])


def _prep1_k(x_ref, w_ref, as_ref, ad_ref, xl_ref, als_ref, ald_ref):
    xl = jnp.dot(x_ref[...], w_ref[...], preferred_element_type=jnp.float32)
    xl_ref[0] = xl[:, :CH]
    xl_ref[1] = xl[:, CH:]
    als_ref[...] = (xl * as_ref[...][None, :]).sum(axis=1)
    ald_ref[...] = (xl * ad_ref[...][None, :]).sum(axis=1)


def _prep2_k(agg_ref, b_ref, w_ref, as_ref, ad_ref, xl_ref, als_ref, ald_ref):
    h = jnp.concatenate([agg_ref[0], agg_ref[1]], axis=1) + b_ref[...][None, :]
    h = jnp.maximum(h, 0.0)
    xl = jnp.dot(h, w_ref[...], preferred_element_type=jnp.float32)
    xl_ref[0] = xl[:, :CH]
    xl_ref[1] = xl[:, CH:]
    als_ref[...] = (xl * as_ref[...][None, :]).sum(axis=1)
    ald_ref[...] = (xl * ad_ref[...][None, :]).sum(axis=1)


def _head_k(agg_ref, b_ref, mw1_ref, mb1_ref, mw2_ref, mb2_ref, o_ref):
    h = jnp.concatenate([agg_ref[0], agg_ref[1]], axis=1) + b_ref[...][None, :]
    h = jnp.maximum(h, 0.0)
    z = jnp.dot(h, mw1_ref[...], preferred_element_type=jnp.float32)
    z = jnp.maximum(z + mb1_ref[...][None, :], 0.0)
    o = jnp.dot(z, mw2_ref[...], preferred_element_type=jnp.float32)
    o_ref[...] = jax.nn.sigmoid(o + mb2_ref[...][None, :])


_node_out = [
    jax.ShapeDtypeStruct((NC, Np, CH), jnp.float32),
    jax.ShapeDtypeStruct((Np,), jnp.float32),
    jax.ShapeDtypeStruct((Np,), jnp.float32),
]


def kernel(x, edge_index, W1, as1, ad1, b1, W2, as2, ad2, b2,
           mw1, mb1, mw2, mb2):
    ei = edge_index.astype(jnp.int32)
    loop = jnp.arange(N, dtype=jnp.int32)
    pad = jnp.full((EPAD,), N, jnp.int32)
    src = jnp.concatenate([ei[0], loop, pad])
    dst = jnp.concatenate([ei[1], loop, pad])
    x_p = jnp.zeros((Np, F_IN), jnp.float32).at[:N].set(x)
    zn = jnp.zeros((Np,), jnp.float32)
    zr = jnp.zeros((STRIPE, CH), jnp.float32)

    xl1, als1, ald1 = pl.pallas_call(
        _prep1_k, out_shape=_node_out)(x_p, W1, as1.reshape(-1), ad1.reshape(-1))
    agg1 = _sc_gat(src, dst, als1, ald1, xl1, zn, zr)

    xl2, als2, ald2 = pl.pallas_call(
        _prep2_k, out_shape=_node_out)(agg1, b1, W2, as2.reshape(-1),
                                       ad2.reshape(-1))
    agg2 = _sc_gat(src, dst, als2, ald2, xl2, zn, zr)

    out = pl.pallas_call(
        _head_k,
        out_shape=jax.ShapeDtypeStruct((Np, NUM_CLASSES), jnp.float32),
    )(agg2, b2, mw1, mb1, mw2, mb2)
    return out[:N]


# speedup vs baseline: 1.1258x; 1.0490x over previous
import functools

import jax
import jax.numpy as jnp
from jax import lax
from jax.experimental import pallas as pl
from jax.experimental.pallas import tpu as pltpu
from jax.experimental.pallas import tpu_sc as plsc

N = 10000
E = 320000
F_IN = 128
C = 256
NUM_CLASSES = 16

NC = 2
NS = 16
L = 16
CH = C // NC
Np = 10240
EL = E + N
B = 80
NBATCH = 258
CHUNK = NBATCH * B
Ep = NS * CHUNK
EPAD = Ep - EL
IR = 6
STRIPE = Np // NS

_mesh = plsc.VectorSubcoreMesh(core_axis_name="c", subcore_axis_name="s")


@functools.partial(
    pl.kernel,
    mesh=_mesh,
    compiler_params=pltpu.CompilerParams(needs_layout_passes=False),
    out_type=jax.ShapeDtypeStruct((NC, Np, CH), jnp.float32),
    scratch_types=[
        pltpu.VMEM((Np,), jnp.float32),
        pltpu.VMEM((Np,), jnp.float32),
        pltpu.VMEM((Np,), jnp.float32),
        pltpu.VMEM((IR, B), jnp.int32),
        pltpu.VMEM((IR, B), jnp.int32),
        pltpu.VMEM((B,), jnp.float32),
        pltpu.VMEM((B, CH), jnp.float32),
        pltpu.VMEM_SHARED((Np,), jnp.float32),
        pltpu.VMEM_SHARED((Np, CH), jnp.float32),
    ] + [pltpu.SemaphoreType.DMA] * (IR + 1),
)
def _sc_gat(src_h, dst_h, als_h, ald_h, xl_h, zn_h, zr_h, agg_h,
            als_v, ald_v, den_v, si_v, di_v, val_v, rows_v,
            den_sh, acc_sh, *sems):
    isem = sems[:IR]
    gsem = sems[IR]
    cid = lax.axis_index("c")
    sid = lax.axis_index("s")
    base_e = sid * CHUNK
    r0 = sid * STRIPE

    pltpu.sync_copy(als_h, als_v)
    pltpu.sync_copy(ald_h, ald_v)
    pltpu.sync_copy(zn_h.at[pl.ds(r0, STRIPE)], den_sh.at[pl.ds(r0, STRIPE)])
    pltpu.sync_copy(zr_h, acc_sh.at[pl.ds(r0, STRIPE)])
    plsc.subcore_barrier()

    def istart(tt, i):
        pltpu.async_copy(src_h.at[pl.ds(base_e + tt * B, B)],
                         si_v.at[i], isem[i])
        pltpu.async_copy(dst_h.at[pl.ds(base_e + tt * B, B)],
                         di_v.at[i], isem[i])

    def iwait(i):
        pltpu.make_async_copy(src_h.at[pl.ds(0, B)], si_v.at[i],
                              isem[i]).wait()
        pltpu.make_async_copy(src_h.at[pl.ds(0, B)], di_v.at[i],
                              isem[i]).wait()

    def _ee(i, k):
        s16 = si_v[i, pl.ds(k, L)]
        d16 = di_v[i, pl.ds(k, L)]
        ee = plsc.load_gather(als_v, [s16]) + plsc.load_gather(ald_v, [d16])
        return jnp.where(ee >= 0.0, ee, ee * 0.2), d16

    def p1_slot(tt, i, prefetch):
        iwait(i)

        @pl.loop(0, B, step=L)
        def _(k):
            ee, _ = _ee(i, k)
            val_v[pl.ds(k, L)] = jnp.exp(ee)

        pltpu.sync_copy(val_v, den_sh.at[di_v.at[i]], add=True)
        if prefetch:
            istart(tt + 3, (i + 3) % IR)

    for tt in range(3):
        istart(tt, tt)

    @pl.loop(0, NBATCH - IR, step=IR)
    def _(t):
        for b6 in range(IR):
            p1_slot(t + b6, b6, True)

    for tt in range(NBATCH - IR, NBATCH):
        p1_slot(tt, tt % IR, tt + 3 < NBATCH)

    plsc.subcore_barrier()
    pltpu.sync_copy(den_sh, den_v)

    def p2_slot(tt, i, prefetch):
        iwait(i)
        cp = pltpu.async_copy(xl_h.at[cid].at[si_v.at[i]], rows_v, gsem)

        @pl.loop(0, B, step=L)
        def _(k):
            ee, d16 = _ee(i, k)
            den16 = plsc.load_gather(den_v, [d16])
            val_v[pl.ds(k, L)] = jnp.exp(ee) / (den16 + 1e-16)

        cp.wait()

        @pl.loop(0, B)
        def _(j):
            a16 = plsc.load_gather(val_v, [jnp.zeros((L,), jnp.int32) + j])
            for f in range(CH // L):
                sl = pl.ds(f * L, L)
                rows_v[j, sl] = rows_v[j, sl] * a16

        pltpu.sync_copy(rows_v, acc_sh.at[di_v.at[i]], add=True)
        if prefetch:
            istart(tt + 3, (i + 3) % IR)

    for tt in range(3):
        istart(tt, tt)

    @pl.loop(0, NBATCH - IR, step=IR)
    def _(t):
        for b6 in range(IR):
            p2_slot(t + b6, b6, True)

    for tt in range(NBATCH - IR, NBATCH):
        p2_slot(tt, tt % IR, tt + 3 < NBATCH)

    plsc.subcore_barrier()
    pltpu.sync_copy(acc_sh.at[pl.ds(r0, STRIPE)],
                    agg_h.at[cid].at[pl.ds(r0, STRIPE)])


def _prep1_k(x_ref, w_ref, as_ref, ad_ref, xl_ref, als_ref, ald_ref):
    xl = jnp.dot(x_ref[...], w_ref[...], preferred_element_type=jnp.float32)
    xl_ref[0] = xl[:, :CH]
    xl_ref[1] = xl[:, CH:]
    als_ref[...] = (xl * as_ref[...][None, :]).sum(axis=1)
    ald_ref[...] = (xl * ad_ref[...][None, :]).sum(axis=1)


def _prep2_k(agg_ref, b_ref, w_ref, as_ref, ad_ref, xl_ref, als_ref, ald_ref):
    h = jnp.concatenate([agg_ref[0], agg_ref[1]], axis=1) + b_ref[...][None, :]
    h = jnp.maximum(h, 0.0)
    xl = jnp.dot(h, w_ref[...], preferred_element_type=jnp.float32)
    xl_ref[0] = xl[:, :CH]
    xl_ref[1] = xl[:, CH:]
    als_ref[...] = (xl * as_ref[...][None, :]).sum(axis=1)
    ald_ref[...] = (xl * ad_ref[...][None, :]).sum(axis=1)


def _head_k(agg_ref, b_ref, mw1_ref, mb1_ref, mw2_ref, mb2_ref, o_ref):
    h = jnp.concatenate([agg_ref[0], agg_ref[1]], axis=1) + b_ref[...][None, :]
    h = jnp.maximum(h, 0.0)
    z = jnp.dot(h, mw1_ref[...], preferred_element_type=jnp.float32)
    z = jnp.maximum(z + mb1_ref[...][None, :], 0.0)
    o = jnp.dot(z, mw2_ref[...], preferred_element_type=jnp.float32)
    o_ref[...] = jax.nn.sigmoid(o + mb2_ref[...][None, :])


_node_out = [
    jax.ShapeDtypeStruct((NC, Np, CH), jnp.float32),
    jax.ShapeDtypeStruct((Np,), jnp.float32),
    jax.ShapeDtypeStruct((Np,), jnp.float32),
]


def kernel(x, edge_index, W1, as1, ad1, b1, W2, as2, ad2, b2,
           mw1, mb1, mw2, mb2):
    ei = edge_index.astype(jnp.int32)
    loop = jnp.arange(N, dtype=jnp.int32)
    pad = jnp.full((EPAD,), N, jnp.int32)
    src = jnp.concatenate([ei[0], loop, pad])
    dst = jnp.concatenate([ei[1], loop, pad])
    x_p = jnp.zeros((Np, F_IN), jnp.float32).at[:N].set(x)
    zn = jnp.zeros((Np,), jnp.float32)
    zr = jnp.zeros((STRIPE, CH), jnp.float32)

    xl1, als1, ald1 = pl.pallas_call(
        _prep1_k, out_shape=_node_out)(x_p, W1, as1.reshape(-1), ad1.reshape(-1))
    agg1 = _sc_gat(src, dst, als1, ald1, xl1, zn, zr)

    xl2, als2, ald2 = pl.pallas_call(
        _prep2_k, out_shape=_node_out)(agg1, b1, W2, as2.reshape(-1),
                                       ad2.reshape(-1))
    agg2 = _sc_gat(src, dst, als2, ald2, xl2, zn, zr)

    out = pl.pallas_call(
        _head_k,
        out_shape=jax.ShapeDtypeStruct((Np, NUM_CLASSES), jnp.float32),
    )(agg2, b2, mw1, mb1, mw2, mb2)
    return out[:N]
